# baseline (speedup 1.0000x reference)
import functools

import jax
import jax.numpy as jnp
from jax import lax
from jax.experimental import pallas as pl
from jax.experimental.pallas import tpu as pltpu
from jax.experimental.pallas import tpu_sc as plsc

NC = 2
NS = 16
LANES = 16
CH = 80


def _sc_mesh():
  return plsc.VectorSubcoreMesh(
      core_axis_name="c", subcore_axis_name="s", num_cores=NC,
      num_subcores=NS)


def _make_gather_pass(E, N, D):
  per_w = E // (NC * NS)
  nch = per_w // CH
  assert nch * CH == per_w and nch >= 4

  def phase(idx_hbm, table, out, idxb, rowsb, gsem, wsem, isem, wid, W):
    def fire_idx(k, j):
      base = wid * per_w + k * CH
      return pltpu.async_copy(idx_hbm.at[pl.ds(base, CH)], idxb.at[j], isem)

    def start_gather(j):
      pltpu.async_copy(table.at[idxb.at[j]], rowsb.at[j], gsem)

    def wait_gather(j):
      pltpu.make_async_copy(table.at[idxb.at[j]], rowsb.at[j], gsem).wait()

    def start_write(k, j):
      base = wid * per_w + k * CH
      pltpu.async_copy(rowsb.at[j], out.at[pl.ds(base, CH)], wsem)

    def wait_write(k, j):
      base = wid * per_w + k * CH
      pltpu.make_async_copy(rowsb.at[j], out.at[pl.ds(base, CH)],
                            wsem).wait()

    fire_idx(0, 0).wait()
    start_gather(0)
    fire_idx(1, 1).wait()

    def stage(k, j, first=False, fire2=True, fire1=True):
      jn, jn2 = (j + 1) % 3, (j + 2) % 3
      wait_gather(j)
      if fire1:
        start_gather(jn)
      if not first:
        wait_write(k - 1, jn2)
      if fire2:
        fire_idx(k + 2, jn2).wait()
      start_write(k, j)

    stage(0, 0, first=True)

    def body(t, _):
      k = 3 * t + 1
      stage(k, 1)
      stage(k + 1, 2)
      stage(k + 2, 0)
      return 0

    nt = (nch - 4) // 3
    lax.fori_loop(0, nt, body, 0)
    for k in range(3 * nt + 1, nch):
      stage(k, k % 3, fire2=(k + 2 < nch), fire1=(k + 1 < nch))
    wait_write(nch - 1, (nch - 1) % 3)

  @functools.partial(
      pl.kernel,
      out_type=[jax.ShapeDtypeStruct((E, 16), jnp.float32),
                jax.ShapeDtypeStruct((E, D), jnp.float32)],
      mesh=_sc_mesh(),
      scratch_types=[
          pltpu.VMEM((3, CH), jnp.int32),
          pltpu.VMEM((3, CH, 16), jnp.float32),
          pltpu.VMEM((3, CH), jnp.int32),
          pltpu.VMEM((3, CH, D), jnp.float32),
          pltpu.SemaphoreType.DMA,
          pltpu.SemaphoreType.DMA,
          pltpu.SemaphoreType.DMA,
      ],
      compiler_params=pltpu.CompilerParams(use_tc_tiling_on_sc=False,
                                           needs_layout_passes=False),
  )
  def kern(xw, x, ei0, ei1, xg, xs, idx16, rows16, idx128, rows128,
           gsem, wsem, isem):
    c = lax.axis_index("c")
    s = lax.axis_index("s")
    wid = s * NC + c
    phase(ei0, xw, xg, idx16, rows16, gsem, wsem, isem, wid, 16)
    phase(ei1, x, xs, idx128, rows128, gsem, wsem, isem, wid, D)

  return kern


def _make_edge_pass(E, N, D, V, indirect):
  vpc = V // NC
  per_s = E // NS
  nch = per_s // CH
  assert nch * CH == per_s and nch >= 5
  stripe = (N // NS) // 8 * 8
  tail = N - NS * stripe

  scratch = [
      pltpu.VMEM((3, CH), jnp.int32),
      pltpu.VMEM((3, CH), jnp.int32),
      pltpu.VMEM((3, CH), jnp.float32),
      pltpu.VMEM((3, CH, D), jnp.float32),
      pltpu.VMEM_SHARED((N, D), jnp.float32),
      pltpu.SemaphoreType.DMA,
      pltpu.SemaphoreType.DMA,
      pltpu.SemaphoreType.DMA,
  ]

  @functools.partial(
      pl.kernel,
      out_type=jax.ShapeDtypeStruct((V, N, D), jnp.float32),
      mesh=_sc_mesh(),
      scratch_types=scratch,
      compiler_params=pltpu.CompilerParams(needs_layout_passes=False),
  )
  def kern(table, ei_src, ei_dst, ewT, init, out,
           sidx, didx, ewv, rows, acc, gsem, ssem, isem):
    c = lax.axis_index("c")
    s = lax.axis_index("s")

    for vi in range(vpc):
      v = c * vpc + vi
      iv = v if indirect else 0
      pltpu.sync_copy(init.at[iv, pl.ds(s * stripe, stripe)],
                      acc.at[pl.ds(s * stripe, stripe)])
      if tail:
        @pl.when(s == 0)
        def _():
          pltpu.sync_copy(init.at[iv, pl.ds(NS * stripe, tail)],
                          acc.at[pl.ds(NS * stripe, tail)])
      plsc.subcore_barrier()

      def fire_smalls(k, j, v=v):
        base = s * per_s + k * CH
        ds = [pltpu.async_copy(ei_dst.at[pl.ds(base, CH)], didx.at[j],
                               isem),
              pltpu.async_copy(ewT.at[pl.ds(v * E + base, CH)], ewv.at[j],
                               isem)]
        if indirect:
          ds.append(pltpu.async_copy(ei_src.at[pl.ds(base, CH)],
                                     sidx.at[j], isem))
        else:
          ds.append(pltpu.async_copy(table.at[pl.ds(base, CH)],
                                     rows.at[j], isem))
        return ds

      def wait_smalls(k, j):
        for d in fire_smalls_desc(k, j):
          d.wait()

      def fire_smalls_desc(k, j, v=v):
        base = s * per_s + k * CH
        ds = [pltpu.make_async_copy(ei_dst.at[pl.ds(base, CH)],
                                    didx.at[j], isem),
              pltpu.make_async_copy(ewT.at[pl.ds(v * E + base, CH)],
                                    ewv.at[j], isem)]
        if indirect:
          ds.append(pltpu.make_async_copy(ei_src.at[pl.ds(base, CH)],
                                          sidx.at[j], isem))
        else:
          ds.append(pltpu.make_async_copy(table.at[pl.ds(base, CH)],
                                          rows.at[j], isem))
        return ds

      def start_gather(j, v=v):
        off = (v * N).astype(jnp.int32)
        for g in range(CH // LANES):
          sl = pl.ds(g * LANES, LANES)
          sidx[j, sl] = sidx[j, sl] + off
        pltpu.async_copy(table.at[sidx.at[j]], rows.at[j], gsem)

      def wait_gather(j):
        pltpu.make_async_copy(table.at[sidx.at[j]], rows.at[j],
                              gsem).wait()

      def compute(j):
        dn = lax.GatherDimensionNumbers(offset_dims=(),
                                        collapsed_slice_dims=(0,),
                                        start_index_map=(0,))

        def grp_body(g16, _):
          ew16 = ewv[j, pl.ds(g16 * LANES, LANES)]

          def lane_body(lane, ew16):
            ewb = lax.gather(
                ew16, jnp.full((LANES, 1), lane, jnp.int32), dn,
                slice_sizes=(1,),
                mode=lax.GatherScatterMode.PROMISE_IN_BOUNDS)
            e = g16 * LANES + lane
            for g in range(D // LANES):
              sl = pl.ds(g * LANES, LANES)
              rows[j, e, sl] = rows[j, e, sl] * ewb
            return ew16

          lax.fori_loop(0, LANES, lane_body, ew16)
          return 0

        lax.fori_loop(0, CH // LANES, grp_body, 0)

      def start_scatter(j):
        pltpu.async_copy(rows.at[j], acc.at[didx.at[j]], ssem, add=True)

      def wait_scatter(j):
        pltpu.make_async_copy(rows.at[j], acc.at[didx.at[j]], ssem).wait()

      def stage(k, j, first=False, fire2=True, fire1=True):
        jn, jn2 = (j + 1) % 3, (j + 2) % 3
        if indirect:
          wait_gather(j)
          if fire1:
            wait_smalls(k + 1, jn)
            start_gather(jn)
          if not first:
            wait_scatter(jn2)
          if fire2:
            fire_smalls(k + 2, jn2)
        else:
          if fire1:
            wait_smalls(k + 1, jn)
          if not first:
            wait_scatter(jn2)
          if fire2:
            fire_smalls(k + 2, jn2)
        compute(j)
        start_scatter(j)

      for d in fire_smalls(0, 0):
        d.wait()
      if indirect:
        start_gather(0)
      fire_smalls(1, 1)
      stage(0, 0, first=True)

      def body(t, _):
        k = 3 * t + 1
        stage(k, 1)
        stage(k + 1, 2)
        stage(k + 2, 0)
        return 0

      nt = (nch - 4) // 3
      lax.fori_loop(0, nt, body, 0)
      for k in range(3 * nt + 1, nch):
        stage(k, k % 3, fire2=(k + 2 < nch), fire1=(k + 1 < nch))
      wait_scatter((nch - 1) % 3)
      plsc.subcore_barrier()
      pltpu.sync_copy(acc.at[pl.ds(s * stripe, stripe)],
                      out.at[v, pl.ds(s * stripe, stripe)])
      if tail:
        @pl.when(s == 0)
        def _():
          pltpu.sync_copy(acc.at[pl.ds(NS * stripe, tail)],
                          out.at[v, pl.ds(NS * stripe, tail)])
      plsc.subcore_barrier()

  return kern


def _tc_xw(x, w_pad):
  N, D = x.shape
  W = w_pad.shape[1]
  BN = 2000

  def body(x_ref, w_ref, o_ref):
    o_ref[...] = jax.lax.dot(x_ref[...], w_ref[...],
                             precision=jax.lax.Precision.HIGHEST)

  return pl.pallas_call(
      body,
      grid=(N // BN,),
      in_specs=[
          pl.BlockSpec((BN, D), lambda i: (i, 0)),
          pl.BlockSpec((D, W), lambda i: (0, 0)),
      ],
      out_specs=pl.BlockSpec((BN, W), lambda i: (i, 0)),
      out_shape=jax.ShapeDtypeStruct((N, W), jnp.float32),
  )(x, w_pad)


def _tc_edge(edge_attr, xg, xs, W_ee, b_ee, W1, b_ew):
  E, DE = edge_attr.shape
  D = W_ee.shape[1]
  V = W1.shape[1]
  BE = 2560

  def body(eattr_ref, xg_ref, xs_ref, wee_ref, bee_ref, w1_ref, bew_ref,
           ea_ref, s_ref, ew_ref, ewT_ref):
    ea = jax.lax.dot(eattr_ref[...], wee_ref[...],
                     precision=jax.lax.Precision.HIGHEST)
    ea = jnp.maximum(ea + bee_ref[...], 0.0)
    ea_ref[...] = ea
    s_ref[...] = ea + xs_ref[...]
    z = jax.lax.dot(ea, w1_ref[...], precision=jax.lax.Precision.HIGHEST)
    z = z + xg_ref[...][:, :V] + bew_ref[...]
    ew = jax.nn.sigmoid(z)
    ew_ref[...] = ew
    ewT_ref[...] = ew.T

  return pl.pallas_call(
      body,
      grid=(E // BE,),
      in_specs=[
          pl.BlockSpec((BE, DE), lambda i: (i, 0)),
          pl.BlockSpec((BE, 16), lambda i: (i, 0)),
          pl.BlockSpec((BE, D), lambda i: (i, 0)),
          pl.BlockSpec((DE, D), lambda i: (0, 0)),
          pl.BlockSpec((D,), lambda i: (0,)),
          pl.BlockSpec((D, V), lambda i: (0, 0)),
          pl.BlockSpec((V,), lambda i: (0,)),
      ],
      out_specs=[
          pl.BlockSpec((BE, D), lambda i: (i, 0)),
          pl.BlockSpec((BE, D), lambda i: (i, 0)),
          pl.BlockSpec((BE, V), lambda i: (i, 0)),
          pl.BlockSpec((V, BE), lambda i: (0, i)),
      ],
      out_shape=[
          jax.ShapeDtypeStruct((E, D), jnp.float32),
          jax.ShapeDtypeStruct((E, D), jnp.float32),
          jax.ShapeDtypeStruct((E, V), jnp.float32),
          jax.ShapeDtypeStruct((V, E), jnp.float32),
      ],
  )(edge_attr, xg, xs, W_ee, b_ee, W1, b_ew)


def _tc_conv(agg, W, b, h_in, emit_out):
  V, N, D = agg.shape
  BN = 2000
  h_bcast = h_in.ndim == 2

  def body(agg_ref, w_ref, b_ref, h_ref, *out_refs):
    a = agg_ref[0]
    o = jax.lax.dot(a, w_ref[0], precision=jax.lax.Precision.HIGHEST)
    o = jnp.maximum(o + b_ref[0], 0.0)
    h = h_ref[...].reshape(BN, D)
    if emit_out:
      out_refs[0][0] = o
      out_refs[1][0] = h + o
    else:
      out_refs[0][0] = h + o

  if h_bcast:
    h_spec = pl.BlockSpec((BN, D), lambda v, i: (i, 0))
  else:
    h_spec = pl.BlockSpec((1, BN, D), lambda v, i: (v, i, 0))

  out_shapes = [jax.ShapeDtypeStruct((V, N, D), jnp.float32)]
  out_specs = [pl.BlockSpec((1, BN, D), lambda v, i: (v, i, 0))]
  if emit_out:
    out_shapes = out_shapes * 2
    out_specs = out_specs * 2
  else:
    out_shapes, out_specs = out_shapes[0], out_specs[0]

  return pl.pallas_call(
      body,
      grid=(V, N // BN),
      in_specs=[
          pl.BlockSpec((1, BN, D), lambda v, i: (v, i, 0)),
          pl.BlockSpec((1, D, D), lambda v, i: (v, 0, 0)),
          pl.BlockSpec((1, 1, D), lambda v, i: (v, 0, 0)),
          h_spec,
      ],
      out_specs=out_specs,
      out_shape=out_shapes,
  )(agg, W, b, h_in)


def kernel(x, edge_index, edge_attr, W_ee, b_ee, W_ew, b_ew, conv_W, conv_b):
  N, D = x.shape
  E = edge_attr.shape[0]
  V = W_ew.shape[1]

  ei0 = edge_index[0]
  ei1 = edge_index[1]

  w2_pad = jnp.zeros((D, 16), jnp.float32).at[:, :V].set(W_ew[D:])
  xw = _tc_xw(x, w2_pad)
  xg, xs = _make_gather_pass(E, N, D)(xw, x, ei0, ei1)

  ea, s_msg, ew, ewT = _tc_edge(edge_attr, xg, xs, W_ee, b_ee, W_ew[:D],
                                b_ew)

  ewT_flat = ewT.reshape(V * E)
  zeros_init = jnp.zeros((1, N, D), jnp.float32)
  pass0 = _make_edge_pass(E, N, D, V, indirect=False)
  agg0 = pass0(s_msg, ei1, ei0, ewT_flat, zeros_init)

  out0, h1 = _tc_conv(agg0, conv_W[0], conv_b[0][:, None, :], x,
                      emit_out=True)

  pass1 = _make_edge_pass(E, N, D, V, indirect=True)
  out0_flat = out0.reshape(V * N, D)
  agg1 = pass1(out0_flat, ei1, ei0, ewT_flat, agg0)

  h2 = _tc_conv(agg1, conv_W[1], conv_b[1][:, None, :], h1, emit_out=False)

  return jnp.transpose(h2, (1, 0, 2)), ew, ea


# speedup vs baseline: 1.4665x; 1.4665x over previous
import functools

import jax
import jax.numpy as jnp
from jax import lax
from jax.experimental import pallas as pl
from jax.experimental.pallas import tpu as pltpu
from jax.experimental.pallas import tpu_sc as plsc

NC = 2
NS = 16
LANES = 16
CH = 80


def _sc_mesh():
  return plsc.VectorSubcoreMesh(
      core_axis_name="c", subcore_axis_name="s", num_cores=NC,
      num_subcores=NS)


def _make_gather_pass(E, N, D):
  per_w = E // (NC * NS)
  nch = per_w // CH
  assert nch * CH == per_w and nch >= 4

  def phase(idx_hbm, table, out, idxb, rowsb, gsem, wsem, isem, wid, W):
    def fire_idx(k, j):
      base = wid * per_w + k * CH
      return pltpu.async_copy(idx_hbm.at[pl.ds(base, CH)], idxb.at[j], isem)

    def start_gather(j):
      pltpu.async_copy(table.at[idxb.at[j]], rowsb.at[j], gsem)

    def wait_gather(j):
      pltpu.make_async_copy(table.at[idxb.at[j]], rowsb.at[j], gsem).wait()

    def start_write(k, j):
      base = wid * per_w + k * CH
      pltpu.async_copy(rowsb.at[j], out.at[pl.ds(base, CH)], wsem)

    def wait_write(k, j):
      base = wid * per_w + k * CH
      pltpu.make_async_copy(rowsb.at[j], out.at[pl.ds(base, CH)],
                            wsem).wait()

    fire_idx(0, 0).wait()
    start_gather(0)
    fire_idx(1, 1).wait()

    def stage(k, j, first=False, fire2=True, fire1=True):
      jn, jn2 = (j + 1) % 3, (j + 2) % 3
      wait_gather(j)
      if fire1:
        start_gather(jn)
      if not first:
        wait_write(k - 1, jn2)
      if fire2:
        fire_idx(k + 2, jn2).wait()
      start_write(k, j)

    stage(0, 0, first=True)

    def body(t, _):
      k = 3 * t + 1
      stage(k, 1)
      stage(k + 1, 2)
      stage(k + 2, 0)
      return 0

    nt = (nch - 4) // 3
    lax.fori_loop(0, nt, body, 0)
    for k in range(3 * nt + 1, nch):
      stage(k, k % 3, fire2=(k + 2 < nch), fire1=(k + 1 < nch))
    wait_write(nch - 1, (nch - 1) % 3)

  @functools.partial(
      pl.kernel,
      out_type=[jax.ShapeDtypeStruct((E, 16), jnp.float32),
                jax.ShapeDtypeStruct((E, D), jnp.float32)],
      mesh=_sc_mesh(),
      scratch_types=[
          pltpu.VMEM((3, CH), jnp.int32),
          pltpu.VMEM((3, CH, 16), jnp.float32),
          pltpu.VMEM((3, CH), jnp.int32),
          pltpu.VMEM((3, CH, D), jnp.float32),
          pltpu.SemaphoreType.DMA,
          pltpu.SemaphoreType.DMA,
          pltpu.SemaphoreType.DMA,
      ],
      compiler_params=pltpu.CompilerParams(use_tc_tiling_on_sc=False,
                                           needs_layout_passes=False),
  )
  def kern(xw, x, ei0, ei1, xg, xs, idx16, rows16, idx128, rows128,
           gsem, wsem, isem):
    c = lax.axis_index("c")
    s = lax.axis_index("s")
    wid = s * NC + c
    phase(ei0, xw, xg, idx16, rows16, gsem, wsem, isem, wid, 16)
    phase(ei1, x, xs, idx128, rows128, gsem, wsem, isem, wid, D)

  return kern


def _make_edge_pass(E, N, D, V, indirect):
  vpc = V // NC
  per_s = E // NS
  nch = per_s // CH
  assert nch * CH == per_s and nch >= 5
  stripe = (N // NS) // 8 * 8
  tail = N - NS * stripe

  scratch = [
      pltpu.VMEM((3, CH), jnp.int32),
      pltpu.VMEM((3, CH), jnp.int32),
      pltpu.VMEM((3, CH), jnp.float32),
      pltpu.VMEM((3, CH, D), jnp.float32),
      pltpu.VMEM_SHARED((N, D), jnp.float32),
      pltpu.SemaphoreType.DMA,
      pltpu.SemaphoreType.DMA,
      pltpu.SemaphoreType.DMA,
  ]

  @functools.partial(
      pl.kernel,
      out_type=jax.ShapeDtypeStruct((V, N, D), jnp.float32),
      mesh=_sc_mesh(),
      scratch_types=scratch,
      compiler_params=pltpu.CompilerParams(needs_layout_passes=False),
  )
  def kern(table, ei_src, ei_dst, ewT, init, out,
           sidx, didx, ewv, rows, acc, gsem, ssem, isem):
    c = lax.axis_index("c")
    s = lax.axis_index("s")

    for vi in range(vpc):
      v = c * vpc + vi
      iv = v if indirect else 0
      pltpu.sync_copy(init.at[iv, pl.ds(s * stripe, stripe)],
                      acc.at[pl.ds(s * stripe, stripe)])
      if tail:
        @pl.when(s == 0)
        def _():
          pltpu.sync_copy(init.at[iv, pl.ds(NS * stripe, tail)],
                          acc.at[pl.ds(NS * stripe, tail)])
      plsc.subcore_barrier()

      def fire_smalls(k, j, v=v):
        base = s * per_s + k * CH
        ds = [pltpu.async_copy(ei_dst.at[pl.ds(base, CH)], didx.at[j],
                               isem),
              pltpu.async_copy(ewT.at[pl.ds(v * E + base, CH)], ewv.at[j],
                               isem)]
        if indirect:
          ds.append(pltpu.async_copy(ei_src.at[pl.ds(base, CH)],
                                     sidx.at[j], isem))
        else:
          ds.append(pltpu.async_copy(table.at[pl.ds(base, CH)],
                                     rows.at[j], isem))
        return ds

      def wait_smalls(k, j):
        for d in fire_smalls_desc(k, j):
          d.wait()

      def fire_smalls_desc(k, j, v=v):
        base = s * per_s + k * CH
        ds = [pltpu.make_async_copy(ei_dst.at[pl.ds(base, CH)],
                                    didx.at[j], isem),
              pltpu.make_async_copy(ewT.at[pl.ds(v * E + base, CH)],
                                    ewv.at[j], isem)]
        if indirect:
          ds.append(pltpu.make_async_copy(ei_src.at[pl.ds(base, CH)],
                                          sidx.at[j], isem))
        else:
          ds.append(pltpu.make_async_copy(table.at[pl.ds(base, CH)],
                                          rows.at[j], isem))
        return ds

      def start_gather(j, v=v):
        off = (v * N).astype(jnp.int32)
        for g in range(CH // LANES):
          sl = pl.ds(g * LANES, LANES)
          sidx[j, sl] = sidx[j, sl] + off
        pltpu.async_copy(table.at[sidx.at[j]], rows.at[j], gsem)

      def wait_gather(j):
        pltpu.make_async_copy(table.at[sidx.at[j]], rows.at[j],
                              gsem).wait()

      def compute(j):
        @functools.partial(plsc.parallel_loop, 0, CH, unroll=4)
        def _(e):
          ewb = plsc.load_gather(ewv.at[j],
                                 [jnp.full((LANES,), e, jnp.int32)])
          for g in range(D // LANES):
            sl = pl.ds(g * LANES, LANES)
            rows[j, e, sl] = rows[j, e, sl] * ewb

      def start_scatter(j):
        pltpu.async_copy(rows.at[j], acc.at[didx.at[j]], ssem, add=True)

      def wait_scatter(j):
        pltpu.make_async_copy(rows.at[j], acc.at[didx.at[j]], ssem).wait()

      def stage(k, j, first=False, fire2=True, fire1=True):
        jn, jn2 = (j + 1) % 3, (j + 2) % 3
        if indirect:
          wait_gather(j)
          if fire1:
            wait_smalls(k + 1, jn)
            start_gather(jn)
          if not first:
            wait_scatter(jn2)
          if fire2:
            fire_smalls(k + 2, jn2)
        else:
          if fire1:
            wait_smalls(k + 1, jn)
          if not first:
            wait_scatter(jn2)
          if fire2:
            fire_smalls(k + 2, jn2)
        compute(j)
        start_scatter(j)

      for d in fire_smalls(0, 0):
        d.wait()
      if indirect:
        start_gather(0)
      fire_smalls(1, 1)
      stage(0, 0, first=True)

      def body(t, _):
        k = 3 * t + 1
        stage(k, 1)
        stage(k + 1, 2)
        stage(k + 2, 0)
        return 0

      nt = (nch - 4) // 3
      lax.fori_loop(0, nt, body, 0)
      for k in range(3 * nt + 1, nch):
        stage(k, k % 3, fire2=(k + 2 < nch), fire1=(k + 1 < nch))
      wait_scatter((nch - 1) % 3)
      plsc.subcore_barrier()
      pltpu.sync_copy(acc.at[pl.ds(s * stripe, stripe)],
                      out.at[v, pl.ds(s * stripe, stripe)])
      if tail:
        @pl.when(s == 0)
        def _():
          pltpu.sync_copy(acc.at[pl.ds(NS * stripe, tail)],
                          out.at[v, pl.ds(NS * stripe, tail)])
      plsc.subcore_barrier()

  return kern


def _tc_xw(x, w_pad):
  N, D = x.shape
  W = w_pad.shape[1]
  BN = 2000

  def body(x_ref, w_ref, o_ref):
    o_ref[...] = jax.lax.dot(x_ref[...], w_ref[...],
                             precision=jax.lax.Precision.HIGHEST)

  return pl.pallas_call(
      body,
      grid=(N // BN,),
      in_specs=[
          pl.BlockSpec((BN, D), lambda i: (i, 0)),
          pl.BlockSpec((D, W), lambda i: (0, 0)),
      ],
      out_specs=pl.BlockSpec((BN, W), lambda i: (i, 0)),
      out_shape=jax.ShapeDtypeStruct((N, W), jnp.float32),
  )(x, w_pad)


def _tc_edge(edge_attr, xg, xs, W_ee, b_ee, W1, b_ew):
  E, DE = edge_attr.shape
  D = W_ee.shape[1]
  V = W1.shape[1]
  BE = 2560

  def body(eattr_ref, xg_ref, xs_ref, wee_ref, bee_ref, w1_ref, bew_ref,
           ea_ref, s_ref, ew_ref, ewT_ref):
    ea = jax.lax.dot(eattr_ref[...], wee_ref[...],
                     precision=jax.lax.Precision.HIGHEST)
    ea = jnp.maximum(ea + bee_ref[...], 0.0)
    ea_ref[...] = ea
    s_ref[...] = ea + xs_ref[...]
    z = jax.lax.dot(ea, w1_ref[...], precision=jax.lax.Precision.HIGHEST)
    z = z + xg_ref[...][:, :V] + bew_ref[...]
    ew = jax.nn.sigmoid(z)
    ew_ref[...] = ew
    ewT_ref[...] = ew.T

  return pl.pallas_call(
      body,
      grid=(E // BE,),
      in_specs=[
          pl.BlockSpec((BE, DE), lambda i: (i, 0)),
          pl.BlockSpec((BE, 16), lambda i: (i, 0)),
          pl.BlockSpec((BE, D), lambda i: (i, 0)),
          pl.BlockSpec((DE, D), lambda i: (0, 0)),
          pl.BlockSpec((D,), lambda i: (0,)),
          pl.BlockSpec((D, V), lambda i: (0, 0)),
          pl.BlockSpec((V,), lambda i: (0,)),
      ],
      out_specs=[
          pl.BlockSpec((BE, D), lambda i: (i, 0)),
          pl.BlockSpec((BE, D), lambda i: (i, 0)),
          pl.BlockSpec((BE, V), lambda i: (i, 0)),
          pl.BlockSpec((V, BE), lambda i: (0, i)),
      ],
      out_shape=[
          jax.ShapeDtypeStruct((E, D), jnp.float32),
          jax.ShapeDtypeStruct((E, D), jnp.float32),
          jax.ShapeDtypeStruct((E, V), jnp.float32),
          jax.ShapeDtypeStruct((V, E), jnp.float32),
      ],
  )(edge_attr, xg, xs, W_ee, b_ee, W1, b_ew)


def _tc_conv(agg, W, b, h_in, emit_out):
  V, N, D = agg.shape
  BN = 2000
  h_bcast = h_in.ndim == 2

  def body(agg_ref, w_ref, b_ref, h_ref, *out_refs):
    a = agg_ref[0]
    o = jax.lax.dot(a, w_ref[0], precision=jax.lax.Precision.HIGHEST)
    o = jnp.maximum(o + b_ref[0], 0.0)
    h = h_ref[...].reshape(BN, D)
    if emit_out:
      out_refs[0][0] = o
      out_refs[1][0] = h + o
    else:
      out_refs[0][0] = h + o

  if h_bcast:
    h_spec = pl.BlockSpec((BN, D), lambda v, i: (i, 0))
  else:
    h_spec = pl.BlockSpec((1, BN, D), lambda v, i: (v, i, 0))

  out_shapes = [jax.ShapeDtypeStruct((V, N, D), jnp.float32)]
  out_specs = [pl.BlockSpec((1, BN, D), lambda v, i: (v, i, 0))]
  if emit_out:
    out_shapes = out_shapes * 2
    out_specs = out_specs * 2
  else:
    out_shapes, out_specs = out_shapes[0], out_specs[0]

  return pl.pallas_call(
      body,
      grid=(V, N // BN),
      in_specs=[
          pl.BlockSpec((1, BN, D), lambda v, i: (v, i, 0)),
          pl.BlockSpec((1, D, D), lambda v, i: (v, 0, 0)),
          pl.BlockSpec((1, 1, D), lambda v, i: (v, 0, 0)),
          h_spec,
      ],
      out_specs=out_specs,
      out_shape=out_shapes,
  )(agg, W, b, h_in)


def kernel(x, edge_index, edge_attr, W_ee, b_ee, W_ew, b_ew, conv_W, conv_b):
  N, D = x.shape
  E = edge_attr.shape[0]
  V = W_ew.shape[1]

  ei0 = edge_index[0]
  ei1 = edge_index[1]

  w2_pad = jnp.zeros((D, 16), jnp.float32).at[:, :V].set(W_ew[D:])
  xw = _tc_xw(x, w2_pad)
  xg, xs = _make_gather_pass(E, N, D)(xw, x, ei0, ei1)

  ea, s_msg, ew, ewT = _tc_edge(edge_attr, xg, xs, W_ee, b_ee, W_ew[:D],
                                b_ew)

  ewT_flat = ewT.reshape(V * E)
  zeros_init = jnp.zeros((1, N, D), jnp.float32)
  pass0 = _make_edge_pass(E, N, D, V, indirect=False)
  agg0 = pass0(s_msg, ei1, ei0, ewT_flat, zeros_init)

  out0, h1 = _tc_conv(agg0, conv_W[0], conv_b[0][:, None, :], x,
                      emit_out=True)

  pass1 = _make_edge_pass(E, N, D, V, indirect=True)
  out0_flat = out0.reshape(V * N, D)
  agg1 = pass1(out0_flat, ei1, ei0, ewT_flat, agg0)

  h2 = _tc_conv(agg1, conv_W[1], conv_b[1][:, None, :], h1, emit_out=False)

  return jnp.transpose(h2, (1, 0, 2)), ew, ea
